# Initial kernel scaffold; baseline (speedup 1.0000x reference)
import jax
import jax.numpy as jnp
from jax.experimental import pallas as pl


def kernel(x, edge_index, batch, W1, b1, W2, b2, w_ih, w_hh, b_ih, b_hh):
    raise NotImplementedError("write your pallas kernel here")


# speedup vs baseline: 16.0391x; 16.0391x over previous
import functools

import jax
import jax.numpy as jnp
from jax import lax
from jax.experimental import pallas as pl
from jax.experimental.pallas import tpu as pltpu
from jax.experimental.pallas import tpu_sc as plsc

N = 10000
E = 320000
D = 128
B = 64
STEPS = 3

NC = 2
NS = 16
NW = NC * NS
EPW = E // NW
CHUNK = 128
NFULL = EPW // CHUNK
REM = EPW - NFULL * CHUNK
NP = 10112
RPS = NP // NS

_MESH = plsc.VectorSubcoreMesh(
    core_axis_name="c", subcore_axis_name="s", num_cores=NC, num_subcores=NS)


def _deg_body(dst_hbm, ones_hbm, zrow_hbm, out_hbm, acc, ones_vm, idx_vm,
              idx_rem_vm):
    c = lax.axis_index("c")
    s = lax.axis_index("s")
    w = c * NS + s

    pltpu.sync_copy(zrow_hbm, acc.at[pl.ds(s * RPS, RPS)])
    pltpu.sync_copy(ones_hbm, ones_vm)
    plsc.subcore_barrier()

    base = w * EPW

    def step(g, carry):
        pltpu.sync_copy(dst_hbm.at[pl.ds(base + g * CHUNK, CHUNK)], idx_vm)
        pltpu.sync_copy(ones_vm, acc.at[idx_vm], add=True)
        return carry

    lax.fori_loop(0, NFULL, step, 0)
    pltpu.sync_copy(dst_hbm.at[pl.ds(base + NFULL * CHUNK, REM)], idx_rem_vm)
    pltpu.sync_copy(ones_vm.at[pl.ds(0, REM)], acc.at[idx_rem_vm], add=True)

    plsc.subcore_barrier()
    pltpu.sync_copy(acc.at[pl.ds(s * RPS, RPS)],
                    out_hbm.at[c].at[pl.ds(s * RPS, RPS)])


@functools.partial(
    pl.kernel,
    out_type=jax.ShapeDtypeStruct((NC, NP, D), jnp.float32),
    mesh=_MESH,
    scratch_types=[
        pltpu.VMEM_SHARED((NP, D), jnp.float32),
        pltpu.VMEM((CHUNK, D), jnp.float32),
        pltpu.VMEM((CHUNK,), jnp.int32),
        pltpu.VMEM((REM,), jnp.int32),
    ],
)
def _deg_sc(dst_hbm, ones_hbm, zrow_hbm, out_hbm, acc, ones_vm, idx_vm,
            idx_rem_vm):
    _deg_body(dst_hbm, ones_hbm, zrow_hbm, out_hbm, acc, ones_vm, idx_vm,
              idx_rem_vm)


def _agg_body(h_hbm, src_hbm, dst_hbm, zfeat_hbm, out_hbm, acc, rows_vm,
              idxs_vm, idxd_vm, rows_rem_vm, idxs_rem_vm, idxd_rem_vm, sem):
    c = lax.axis_index("c")
    s = lax.axis_index("s")
    w = c * NS + s

    pltpu.sync_copy(zfeat_hbm, acc.at[pl.ds(s * RPS, RPS)])
    plsc.subcore_barrier()

    base = w * EPW

    def step(g, carry):
        e0 = base + g * CHUNK
        pltpu.sync_copy(src_hbm.at[pl.ds(e0, CHUNK)], idxs_vm)
        pltpu.sync_copy(dst_hbm.at[pl.ds(e0, CHUNK)], idxd_vm)
        pltpu.async_copy(h_hbm.at[idxs_vm], rows_vm, sem).wait()
        pltpu.sync_copy(rows_vm, acc.at[idxd_vm], add=True)
        return carry

    lax.fori_loop(0, NFULL, step, 0)
    e0 = base + NFULL * CHUNK
    pltpu.sync_copy(src_hbm.at[pl.ds(e0, REM)], idxs_rem_vm)
    pltpu.sync_copy(dst_hbm.at[pl.ds(e0, REM)], idxd_rem_vm)
    pltpu.async_copy(h_hbm.at[idxs_rem_vm], rows_rem_vm, sem).wait()
    pltpu.sync_copy(rows_rem_vm, acc.at[idxd_rem_vm], add=True)

    plsc.subcore_barrier()
    pltpu.sync_copy(acc.at[pl.ds(s * RPS, RPS)],
                    out_hbm.at[c].at[pl.ds(s * RPS, RPS)])


@functools.partial(
    pl.kernel,
    out_type=jax.ShapeDtypeStruct((NC, NP, D), jnp.float32),
    mesh=_MESH,
    scratch_types=[
        pltpu.VMEM_SHARED((NP, D), jnp.float32),
        pltpu.VMEM((CHUNK, D), jnp.float32),
        pltpu.VMEM((CHUNK,), jnp.int32),
        pltpu.VMEM((CHUNK,), jnp.int32),
        pltpu.VMEM((REM, D), jnp.float32),
        pltpu.VMEM((REM,), jnp.int32),
        pltpu.VMEM((REM,), jnp.int32),
        pltpu.SemaphoreType.DMA,
    ],
)
def _agg_sc(h_hbm, src_hbm, dst_hbm, zfeat_hbm, out_hbm, acc, rows_vm,
            idxs_vm, idxd_vm, rows_rem_vm, idxs_rem_vm, idxd_rem_vm, sem):
    _agg_body(h_hbm, src_hbm, dst_hbm, zfeat_hbm, out_hbm, acc, rows_vm,
              idxs_vm, idxd_vm, rows_rem_vm, idxs_rem_vm, idxd_rem_vm, sem)


def _dinv_from(degp):
    deg = degp[0, :N, 0:1] + degp[1, :N, 0:1] + 1.0
    return lax.rsqrt(deg)


def _k1_body(x_ref, w_ref, degp_ref, o_ref):
    dinv = _dinv_from(degp_ref[...])
    h = jnp.dot(x_ref[...], w_ref[...], preferred_element_type=jnp.float32)
    o_ref[...] = h * dinv


def _k1(x, W1, degp):
    return pl.pallas_call(
        _k1_body,
        out_shape=jax.ShapeDtypeStruct((N, D), jnp.float32),
    )(x, W1, degp)


def _k2_body(p_ref, hs1_ref, degp_ref, b1_ref, w2_ref, o_ref):
    dinv = _dinv_from(degp_ref[...])
    tot = p_ref[0, :N] + p_ref[1, :N] + hs1_ref[...]
    h1 = jnp.maximum(tot * dinv + b1_ref[...], 0.0)
    h2 = jnp.dot(h1, w2_ref[...], preferred_element_type=jnp.float32)
    o_ref[...] = h2 * dinv


def _k2(p, hs1, degp, b1, W2):
    return pl.pallas_call(
        _k2_body,
        out_shape=jax.ShapeDtypeStruct((N, D), jnp.float32),
    )(p, hs1, degp, b1.reshape(1, D), W2)


def _k3_body(p_ref, hs2_ref, degp_ref, b2_ref, batch_ref, wih_ref, whh_ref,
             bih_ref, bhh_ref, o_ref):
    dinv = _dinv_from(degp_ref[...])
    h2 = (p_ref[0, :N] + p_ref[1, :N] + hs2_ref[...]) * dinv + b2_ref[...]

    seg = batch_ref[0:1, :]
    bids = lax.broadcasted_iota(jnp.int32, (B, N), 0)
    member = bids == seg

    q_star = jnp.zeros((B, 2 * D), jnp.float32)
    h = jnp.zeros((B, D), jnp.float32)
    c = jnp.zeros((B, D), jnp.float32)
    wih = wih_ref[...]
    whh = whh_ref[...]
    bias = bih_ref[...] + bhh_ref[...]

    for _ in range(STEPS):
        gates = (
            lax.dot_general(q_star, wih, (((1,), (1,)), ((), ())),
                            preferred_element_type=jnp.float32)
            + lax.dot_general(h, whh, (((1,), (1,)), ((), ())),
                              preferred_element_type=jnp.float32)
            + bias)
        ig = jax.nn.sigmoid(gates[:, 0:D])
        fg = jax.nn.sigmoid(gates[:, D:2 * D])
        gg = jnp.tanh(gates[:, 2 * D:3 * D])
        og = jax.nn.sigmoid(gates[:, 3 * D:4 * D])
        c = fg * c + ig * gg
        h = og * jnp.tanh(c)

        e = lax.dot_general(h, h2, (((1,), (1,)), ((), ())),
                            preferred_element_type=jnp.float32)
        em = jnp.where(member, e, -jnp.inf)
        emax = jnp.max(em, axis=1, keepdims=True)
        emax = jnp.where(emax < -3e38, 0.0, emax)
        a = jnp.exp(em - emax)
        ssum = jnp.sum(a, axis=1, keepdims=True)
        r = lax.dot_general(a, h2, (((1,), (0,)), ((), ())),
                            preferred_element_type=jnp.float32)
        r = r / (ssum + 1e-16)
        q_star = jnp.concatenate([h, r], axis=1)

    o_ref[...] = q_star


def _k3(p2, hs2, degp, b2, batch, w_ih, w_hh, b_ih, b_hh):
    batch8 = jnp.broadcast_to(batch[None, :], (8, N))
    return pl.pallas_call(
        _k3_body,
        out_shape=jax.ShapeDtypeStruct((B, 2 * D), jnp.float32),
    )(p2, hs2, degp, b2.reshape(1, D), batch8, w_ih, w_hh,
      b_ih.reshape(1, 4 * D), b_hh.reshape(1, 4 * D))


def kernel(x, edge_index, batch, W1, b1, W2, b2, w_ih, w_hh, b_ih, b_hh):
    src = edge_index[0]
    dst = edge_index[1]

    onehot = jnp.zeros((CHUNK, D), jnp.float32).at[:, 0].set(1.0)
    zfeat = jnp.zeros((RPS, D), jnp.float32)

    degp = _deg_sc(dst, onehot, zfeat)
    hs1 = _k1(x, W1, degp)
    p1 = _agg_sc(hs1, src, dst, zfeat)
    hs2 = _k2(p1, hs1, degp, b1, W2)
    p2 = _agg_sc(hs2, src, dst, zfeat)
    return _k3(p2, hs2, degp, b2, batch, w_ih, w_hh, b_ih, b_hh)


# speedup vs baseline: 22.5190x; 1.4040x over previous
import functools

import jax
import jax.numpy as jnp
from jax import lax
from jax.experimental import pallas as pl
from jax.experimental.pallas import tpu as pltpu
from jax.experimental.pallas import tpu_sc as plsc

N = 10000
E = 320000
D = 128
B = 64
STEPS = 3

NC = 2
NS = 16
NW = NC * NS
EPW = E // NW
CHUNK = 128
NFULL = EPW // CHUNK
REM = EPW - NFULL * CHUNK
NP = 10112
RPS = NP // NS

_MESH = plsc.VectorSubcoreMesh(
    core_axis_name="c", subcore_axis_name="s", num_cores=NC, num_subcores=NS)


def _deg_body(dst_hbm, ones_hbm, zrow_hbm, out_hbm, acc, ones_vm, idx_vm,
              idx_rem_vm):
    c = lax.axis_index("c")
    s = lax.axis_index("s")
    w = c * NS + s

    pltpu.sync_copy(zrow_hbm, acc.at[pl.ds(s * RPS, RPS)])
    pltpu.sync_copy(ones_hbm, ones_vm)
    plsc.subcore_barrier()

    base = w * EPW

    def step(g, carry):
        pltpu.sync_copy(dst_hbm.at[pl.ds(base + g * CHUNK, CHUNK)], idx_vm)
        pltpu.sync_copy(ones_vm, acc.at[idx_vm], add=True)
        return carry

    lax.fori_loop(0, NFULL, step, 0)
    pltpu.sync_copy(dst_hbm.at[pl.ds(base + NFULL * CHUNK, REM)], idx_rem_vm)
    pltpu.sync_copy(ones_vm.at[pl.ds(0, REM)], acc.at[idx_rem_vm], add=True)

    plsc.subcore_barrier()
    pltpu.sync_copy(acc.at[pl.ds(s * RPS, RPS)],
                    out_hbm.at[c].at[pl.ds(s * RPS, RPS)])


@functools.partial(
    pl.kernel,
    out_type=jax.ShapeDtypeStruct((NC, NP, D), jnp.float32),
    mesh=_MESH,
    scratch_types=[
        pltpu.VMEM_SHARED((NP, D), jnp.float32),
        pltpu.VMEM((CHUNK, D), jnp.float32),
        pltpu.VMEM((CHUNK,), jnp.int32),
        pltpu.VMEM((REM,), jnp.int32),
    ],
)
def _deg_sc(dst_hbm, ones_hbm, zrow_hbm, out_hbm, acc, ones_vm, idx_vm,
            idx_rem_vm):
    _deg_body(dst_hbm, ones_hbm, zrow_hbm, out_hbm, acc, ones_vm, idx_vm,
              idx_rem_vm)


def _agg_body(h_hbm, src_hbm, dst_hbm, zfeat_hbm, out_hbm, acc,
              rows0, rows1, idxs0, idxs1, idxd0, idxd1,
              rows_rem_vm, idxs_rem_vm, idxd_rem_vm, sem0, sem1, semr):
    c = lax.axis_index("c")
    s = lax.axis_index("s")
    w = c * NS + s
    base = w * EPW

    rows = (rows0, rows1)
    idxs = (idxs0, idxs1)
    idxd = (idxd0, idxd1)
    sems = (sem0, sem1)

    def fire(g, b):
        e0 = base + g * CHUNK
        pltpu.sync_copy(src_hbm.at[pl.ds(e0, CHUNK)], idxs[b])
        pltpu.sync_copy(dst_hbm.at[pl.ds(e0, CHUNK)], idxd[b])
        pltpu.async_copy(h_hbm.at[idxs[b]], rows[b], sems[b])

    def drain_scatter(b):
        pltpu.make_async_copy(h_hbm.at[idxs[b]], rows[b], sems[b]).wait()
        pltpu.sync_copy(rows[b], acc.at[idxd[b]], add=True)

    pltpu.sync_copy(zfeat_hbm, acc.at[pl.ds(s * RPS, RPS)])
    fire(0, 0)
    plsc.subcore_barrier()

    @pl.loop(0, (NFULL - 2) // 2)
    def _pipeline(i):
        g = 2 * i
        fire(g + 1, 1)
        drain_scatter(0)
        fire(g + 2, 0)
        drain_scatter(1)

    fire(NFULL - 1, 1)
    drain_scatter(0)
    e0 = base + NFULL * CHUNK
    pltpu.sync_copy(src_hbm.at[pl.ds(e0, REM)], idxs_rem_vm)
    pltpu.sync_copy(dst_hbm.at[pl.ds(e0, REM)], idxd_rem_vm)
    pltpu.async_copy(h_hbm.at[idxs_rem_vm], rows_rem_vm, semr)
    drain_scatter(1)
    pltpu.make_async_copy(h_hbm.at[idxs_rem_vm], rows_rem_vm, semr).wait()
    pltpu.sync_copy(rows_rem_vm, acc.at[idxd_rem_vm], add=True)

    plsc.subcore_barrier()
    pltpu.sync_copy(acc.at[pl.ds(s * RPS, RPS)],
                    out_hbm.at[c].at[pl.ds(s * RPS, RPS)])


@functools.partial(
    pl.kernel,
    out_type=jax.ShapeDtypeStruct((NC, NP, D), jnp.float32),
    mesh=_MESH,
    scratch_types=[
        pltpu.VMEM_SHARED((NP, D), jnp.float32),
        pltpu.VMEM((CHUNK, D), jnp.float32),
        pltpu.VMEM((CHUNK, D), jnp.float32),
        pltpu.VMEM((CHUNK,), jnp.int32),
        pltpu.VMEM((CHUNK,), jnp.int32),
        pltpu.VMEM((CHUNK,), jnp.int32),
        pltpu.VMEM((CHUNK,), jnp.int32),
        pltpu.VMEM((REM, D), jnp.float32),
        pltpu.VMEM((REM,), jnp.int32),
        pltpu.VMEM((REM,), jnp.int32),
        pltpu.SemaphoreType.DMA,
        pltpu.SemaphoreType.DMA,
        pltpu.SemaphoreType.DMA,
    ],
)
def _agg_sc(h_hbm, src_hbm, dst_hbm, zfeat_hbm, out_hbm, acc,
            rows0, rows1, idxs0, idxs1, idxd0, idxd1,
            rows_rem_vm, idxs_rem_vm, idxd_rem_vm, sem0, sem1, semr):
    _agg_body(h_hbm, src_hbm, dst_hbm, zfeat_hbm, out_hbm, acc,
              rows0, rows1, idxs0, idxs1, idxd0, idxd1,
              rows_rem_vm, idxs_rem_vm, idxd_rem_vm, sem0, sem1, semr)


def _dinv_from(degp):
    deg = degp[0, :N, 0:1] + degp[1, :N, 0:1] + 1.0
    return lax.rsqrt(deg)


def _k1_body(x_ref, w_ref, degp_ref, o_ref):
    dinv = _dinv_from(degp_ref[...])
    h = jnp.dot(x_ref[...], w_ref[...], preferred_element_type=jnp.float32)
    o_ref[...] = h * dinv


def _k1(x, W1, degp):
    return pl.pallas_call(
        _k1_body,
        out_shape=jax.ShapeDtypeStruct((N, D), jnp.float32),
    )(x, W1, degp)


def _k2_body(p_ref, hs1_ref, degp_ref, b1_ref, w2_ref, o_ref):
    dinv = _dinv_from(degp_ref[...])
    tot = p_ref[0, :N] + p_ref[1, :N] + hs1_ref[...]
    h1 = jnp.maximum(tot * dinv + b1_ref[...], 0.0)
    h2 = jnp.dot(h1, w2_ref[...], preferred_element_type=jnp.float32)
    o_ref[...] = h2 * dinv


def _k2(p, hs1, degp, b1, W2):
    return pl.pallas_call(
        _k2_body,
        out_shape=jax.ShapeDtypeStruct((N, D), jnp.float32),
    )(p, hs1, degp, b1.reshape(1, D), W2)


def _k3_body(p_ref, hs2_ref, degp_ref, b2_ref, batch_ref, wih_ref, whh_ref,
             bih_ref, bhh_ref, o_ref):
    dinv = _dinv_from(degp_ref[...])
    h2 = (p_ref[0, :N] + p_ref[1, :N] + hs2_ref[...]) * dinv + b2_ref[...]

    seg = batch_ref[0:1, :]
    bids = lax.broadcasted_iota(jnp.int32, (B, N), 0)
    member = bids == seg

    q_star = jnp.zeros((B, 2 * D), jnp.float32)
    h = jnp.zeros((B, D), jnp.float32)
    c = jnp.zeros((B, D), jnp.float32)
    wih = wih_ref[...]
    whh = whh_ref[...]
    bias = bih_ref[...] + bhh_ref[...]

    for _ in range(STEPS):
        gates = (
            lax.dot_general(q_star, wih, (((1,), (1,)), ((), ())),
                            preferred_element_type=jnp.float32)
            + lax.dot_general(h, whh, (((1,), (1,)), ((), ())),
                              preferred_element_type=jnp.float32)
            + bias)
        ig = jax.nn.sigmoid(gates[:, 0:D])
        fg = jax.nn.sigmoid(gates[:, D:2 * D])
        gg = jnp.tanh(gates[:, 2 * D:3 * D])
        og = jax.nn.sigmoid(gates[:, 3 * D:4 * D])
        c = fg * c + ig * gg
        h = og * jnp.tanh(c)

        e = lax.dot_general(h, h2, (((1,), (1,)), ((), ())),
                            preferred_element_type=jnp.float32)
        em = jnp.where(member, e, -jnp.inf)
        emax = jnp.max(em, axis=1, keepdims=True)
        emax = jnp.where(emax < -3e38, 0.0, emax)
        a = jnp.exp(em - emax)
        ssum = jnp.sum(a, axis=1, keepdims=True)
        r = lax.dot_general(a, h2, (((1,), (0,)), ((), ())),
                            preferred_element_type=jnp.float32)
        r = r / (ssum + 1e-16)
        q_star = jnp.concatenate([h, r], axis=1)

    o_ref[...] = q_star


def _k3(p2, hs2, degp, b2, batch, w_ih, w_hh, b_ih, b_hh):
    batch8 = jnp.broadcast_to(batch[None, :], (8, N))
    return pl.pallas_call(
        _k3_body,
        out_shape=jax.ShapeDtypeStruct((B, 2 * D), jnp.float32),
    )(p2, hs2, degp, b2.reshape(1, D), batch8, w_ih, w_hh,
      b_ih.reshape(1, 4 * D), b_hh.reshape(1, 4 * D))


def kernel(x, edge_index, batch, W1, b1, W2, b2, w_ih, w_hh, b_ih, b_hh):
    src = edge_index[0]
    dst = edge_index[1]

    onehot = jnp.zeros((CHUNK, D), jnp.float32).at[:, 0].set(1.0)
    zfeat = jnp.zeros((RPS, D), jnp.float32)

    degp = _deg_sc(dst, onehot, zfeat)
    hs1 = _k1(x, W1, degp)
    p1 = _agg_sc(hs1, src, dst, zfeat)
    hs2 = _k2(p1, hs1, degp, b1, W2)
    p2 = _agg_sc(hs2, src, dst, zfeat)
    return _k3(p2, hs2, degp, b2, batch, w_ih, w_hh, b_ih, b_hh)
